# baseline (speedup 1.0000x reference)
import functools

import numpy as np

import jax
import jax.numpy as jnp
from jax.experimental import pallas as pl
from jax.experimental.pallas import tpu as pltpu
from jax.experimental.pallas import tpu_sc as plsc

_P = 16
_B = 16
_H = 512
_W = 512
_HP = _H // _P
_WP = _W // _P
_NP = _HP * _WP
_TBL = 1040

_U32 = np.uint64(0xFFFFFFFF)
_ROT = ((13, 15, 26, 6), (17, 29, 16, 24))


def _threefry2x32(k0, k1, x0, x1):
    ks0 = np.uint64(k0) & _U32
    ks1 = np.uint64(k1) & _U32
    ks2 = ks0 ^ ks1 ^ np.uint64(0x1BD11BDA)
    x0 = (np.asarray(x0, np.uint64) + ks0) & _U32
    x1 = (np.asarray(x1, np.uint64) + ks1) & _U32
    sched = ((ks1, ks2), (ks2, ks0), (ks0, ks1), (ks1, ks2), (ks2, ks0))
    for r in range(5):
        for d in _ROT[r % 2]:
            x0 = (x0 + x1) & _U32
            x1 = (((x1 << np.uint64(d)) | (x1 >> np.uint64(32 - d))) & _U32) ^ x0
        a, b = sched[r]
        x0 = (x0 + a) & _U32
        x1 = (x1 + b + np.uint64(r + 1)) & _U32
    return x0, x1


def _build_draw_table():
    tbl = np.zeros((_B, _TBL), np.int32)
    spans = np.maximum(np.arange(_TBL, dtype=np.uint64), 1)
    for i in range(_B):
        ki = _threefry2x32(0, 42, np.uint64(0), np.uint64(i))
        y0, y1 = _threefry2x32(ki[0], ki[1], np.array([0, 0], np.uint64),
                               np.array([0, 1], np.uint64))
        sub1, sub2 = (y0[0], y1[0]), (y0[1], y1[1])
        hi0, hi1 = _threefry2x32(sub1[0], sub1[1], np.uint64(0), np.uint64(0))
        lo0, lo1 = _threefry2x32(sub2[0], sub2[1], np.uint64(0), np.uint64(0))
        higher = np.uint64(hi0 ^ hi1)
        lower = np.uint64(lo0 ^ lo1)
        mult = (np.uint64(2 ** 16) % spans)
        mult = (mult * mult) % spans
        off = ((higher % spans) * mult + (lower % spans)) % spans
        tbl[i] = off.astype(np.int32)
    return tbl


_DRAW_TABLE = _build_draw_table()


def _pool_kernel(mask_ref, out_ref):
    x = mask_ref[0, 0]

    gi = jax.lax.broadcasted_iota(jnp.int32, (_HP, _H), 0)
    ci = jax.lax.broadcasted_iota(jnp.int32, (_HP, _H), 1)
    rowpool = (ci // _P == gi).astype(jnp.float32)
    cj = jax.lax.broadcasted_iota(jnp.int32, (_W, _WP), 0)
    gj = jax.lax.broadcasted_iota(jnp.int32, (_W, _WP), 1)
    colpool = (cj // _P == gj).astype(jnp.float32)

    hp = jax.lax.Precision.HIGHEST
    a = jnp.dot(rowpool, x, precision=hp)
    out_ref[0] = jnp.dot(a, colpool, precision=hp)


_sc_mesh = plsc.VectorSubcoreMesh(core_axis_name="c", subcore_axis_name="s")


@functools.partial(
    pl.kernel,
    mesh=_sc_mesh,
    compiler_params=pltpu.CompilerParams(needs_layout_passes=False),
    out_type=jax.ShapeDtypeStruct((_B, 16), jnp.int32),
    scratch_types=[
        pltpu.VMEM((_HP, _WP), jnp.float32),
        pltpu.VMEM((_TBL,), jnp.int32),
        pltpu.VMEM((16,), jnp.int32),
    ],
)
def _sc_select_kernel(res_hbm, tbl_hbm, out_hbm, res_v, tbl_v, out_v):
    c = jax.lax.axis_index("c")
    s = jax.lax.axis_index("s")

    @pl.when(c == 0)
    def _():
        b = s
        pltpu.sync_copy(res_hbm.at[b], res_v)
        pltpu.sync_copy(tbl_hbm.at[b], tbl_v)

        thresh = jnp.full((16,), float(_P * _P), jnp.float32)
        lanes = jax.lax.iota(jnp.int32, 16)

        one = jnp.full((16,), 1, jnp.int32)
        zero = jnp.full((16,), 0, jnp.int32)

        n = jnp.int32(0)
        for r in range(_HP):
            for p in range(2):
                v = res_v[r, pl.ds(p * 16, 16)]
                n = n + jnp.sum(jnp.where(v < thresh, one, zero))

        jv = plsc.load_gather(tbl_v, [jnp.full((16,), n, jnp.int32)])
        t = jnp.max(jv) + 1

        run = jnp.int32(0)
        flat = jnp.int32(0)
        for r in range(_HP):
            for p in range(2):
                v = res_v[r, pl.ds(p * 16, 16)]
                mi = jnp.where(v < thresh, one, zero)
                cs = jax.lax.cumsum(mi)
                cnt = jnp.sum(mi)
                sel = jnp.where(mi == one, cs, zero) == (t - run)
                pos = jnp.min(jnp.where(sel, lanes, 16))
                hit = (run < t) & (t <= run + cnt)
                flat = jnp.where(hit, (r * 2 + p) * 16 + pos, flat)
                run = run + cnt

        h = (flat >> 5) << 4
        w = (flat & 31) << 4
        out_v[...] = jnp.where(lanes == 0, h, jnp.where(lanes == 1, w, 0))
        pltpu.sync_copy(out_v, out_hbm.at[b])


@jax.jit
def kernel(mask):
    res = pl.pallas_call(
        _pool_kernel,
        grid=(_B,),
        in_specs=[pl.BlockSpec((1, 1, _H, _W), lambda i: (i, 0, 0, 0))],
        out_specs=pl.BlockSpec((1, _HP, _WP), lambda i: (i, 0, 0)),
        out_shape=jax.ShapeDtypeStruct((_B, _HP, _WP), jnp.float32),
    )(mask)
    out = _sc_select_kernel(res, _DRAW_TABLE)
    return out[:, 0], out[:, 1]


# speedup vs baseline: 1.0728x; 1.0728x over previous
import functools

import numpy as np

import jax
import jax.numpy as jnp
from jax.experimental import pallas as pl
from jax.experimental.pallas import tpu as pltpu
from jax.experimental.pallas import tpu_sc as plsc

_P = 16
_B = 16
_H = 512
_W = 512
_HP = _H // _P
_WP = _W // _P
_NP = _HP * _WP
_TBL = 1040

_U32 = np.uint64(0xFFFFFFFF)
_ROT = ((13, 15, 26, 6), (17, 29, 16, 24))


def _threefry2x32(k0, k1, x0, x1):
    ks0 = np.uint64(k0) & _U32
    ks1 = np.uint64(k1) & _U32
    ks2 = ks0 ^ ks1 ^ np.uint64(0x1BD11BDA)
    x0 = (np.asarray(x0, np.uint64) + ks0) & _U32
    x1 = (np.asarray(x1, np.uint64) + ks1) & _U32
    sched = ((ks1, ks2), (ks2, ks0), (ks0, ks1), (ks1, ks2), (ks2, ks0))
    for r in range(5):
        for d in _ROT[r % 2]:
            x0 = (x0 + x1) & _U32
            x1 = (((x1 << np.uint64(d)) | (x1 >> np.uint64(32 - d))) & _U32) ^ x0
        a, b = sched[r]
        x0 = (x0 + a) & _U32
        x1 = (x1 + b + np.uint64(r + 1)) & _U32
    return x0, x1


def _build_draw_table():
    tbl = np.zeros((_B, _TBL), np.int32)
    spans = np.maximum(np.arange(_TBL, dtype=np.uint64), 1)
    for i in range(_B):
        ki = _threefry2x32(0, 42, np.uint64(0), np.uint64(i))
        y0, y1 = _threefry2x32(ki[0], ki[1], np.array([0, 0], np.uint64),
                               np.array([0, 1], np.uint64))
        sub1, sub2 = (y0[0], y1[0]), (y0[1], y1[1])
        hi0, hi1 = _threefry2x32(sub1[0], sub1[1], np.uint64(0), np.uint64(0))
        lo0, lo1 = _threefry2x32(sub2[0], sub2[1], np.uint64(0), np.uint64(0))
        higher = np.uint64(hi0 ^ hi1)
        lower = np.uint64(lo0 ^ lo1)
        mult = (np.uint64(2 ** 16) % spans)
        mult = (mult * mult) % spans
        off = ((higher % spans) * mult + (lower % spans)) % spans
        tbl[i] = off.astype(np.int32)
    return tbl


_DRAW_TABLE = _build_draw_table()


_NBUF = 4


def _pool_kernel(mask_hbm, out_ref, *args):
    bufs = args[:_NBUF]
    sems = args[_NBUF:]

    gi = jax.lax.broadcasted_iota(jnp.int32, (_HP, _H), 0)
    ci = jax.lax.broadcasted_iota(jnp.int32, (_HP, _H), 1)
    rowpool = (ci // _P == gi).astype(jnp.float32)
    cj = jax.lax.broadcasted_iota(jnp.int32, (_W, _WP), 0)
    gj = jax.lax.broadcasted_iota(jnp.int32, (_W, _WP), 1)
    colpool = (cj // _P == gj).astype(jnp.float32)
    hp = jax.lax.Precision.HIGHEST

    def _start(b, j):
        pltpu.make_async_copy(mask_hbm.at[b, 0], bufs[j], sems[j]).start()

    for j in range(_NBUF):
        _start(j, j)

    def body(g, _):
        for j in range(_NBUF):
            b = g * _NBUF + j
            pltpu.make_async_copy(mask_hbm.at[b, 0], bufs[j], sems[j]).wait()
            x = bufs[j][...]
            a = jnp.dot(rowpool, x, precision=hp)
            out_ref[pl.ds(b, 1)] = jnp.dot(a, colpool,
                                           precision=hp)[None]

            @pl.when(g < _B // _NBUF - 1)
            def _():
                _start(b + _NBUF, j)
        return 0

    jax.lax.fori_loop(0, _B // _NBUF, body, 0)


_sc_mesh = plsc.VectorSubcoreMesh(core_axis_name="c", subcore_axis_name="s")


@functools.partial(
    pl.kernel,
    mesh=_sc_mesh,
    compiler_params=pltpu.CompilerParams(needs_layout_passes=False),
    out_type=jax.ShapeDtypeStruct((_B, 16), jnp.int32),
    scratch_types=[
        pltpu.VMEM((_HP, _WP), jnp.float32),
        pltpu.VMEM((_TBL,), jnp.int32),
        pltpu.VMEM((16,), jnp.int32),
    ],
)
def _sc_select_kernel(res_hbm, tbl_hbm, out_hbm, res_v, tbl_v, out_v):
    c = jax.lax.axis_index("c")
    s = jax.lax.axis_index("s")

    @pl.when(c == 0)
    def _():
        b = s
        pltpu.sync_copy(res_hbm.at[b], res_v)
        pltpu.sync_copy(tbl_hbm.at[b], tbl_v)

        thresh = jnp.full((16,), float(_P * _P), jnp.float32)
        lanes = jax.lax.iota(jnp.int32, 16)

        one = jnp.full((16,), 1, jnp.int32)
        zero = jnp.full((16,), 0, jnp.int32)

        n = jnp.int32(0)
        for r in range(_HP):
            for p in range(2):
                v = res_v[r, pl.ds(p * 16, 16)]
                n = n + jnp.sum(jnp.where(v < thresh, one, zero))

        jv = plsc.load_gather(tbl_v, [jnp.full((16,), n, jnp.int32)])
        t = jnp.max(jv) + 1

        run = jnp.int32(0)
        flat = jnp.int32(0)
        for r in range(_HP):
            for p in range(2):
                v = res_v[r, pl.ds(p * 16, 16)]
                mi = jnp.where(v < thresh, one, zero)
                cs = jax.lax.cumsum(mi)
                cnt = jnp.sum(mi)
                sel = jnp.where(mi == one, cs, zero) == (t - run)
                pos = jnp.min(jnp.where(sel, lanes, 16))
                hit = (run < t) & (t <= run + cnt)
                flat = jnp.where(hit, (r * 2 + p) * 16 + pos, flat)
                run = run + cnt

        h = (flat >> 5) << 4
        w = (flat & 31) << 4
        out_v[...] = jnp.where(lanes == 0, h, jnp.where(lanes == 1, w, 0))
        pltpu.sync_copy(out_v, out_hbm.at[b])


@jax.jit
def kernel(mask):
    res = pl.pallas_call(
        _pool_kernel,
        in_specs=[pl.BlockSpec(memory_space=pl.ANY)],
        out_specs=pl.BlockSpec(memory_space=pltpu.VMEM),
        out_shape=jax.ShapeDtypeStruct((_B, _HP, _WP), jnp.float32),
        scratch_shapes=(
            [pltpu.VMEM((_H, _W), jnp.float32) for _ in range(_NBUF)]
            + [pltpu.SemaphoreType.DMA for _ in range(_NBUF)]
        ),
    )(mask)
    out = _sc_select_kernel(res, _DRAW_TABLE)
    return out[:, 0], out[:, 1]
